# baseline (speedup 1.0000x reference)
import jax
import jax.numpy as jnp
from jax import lax
from jax.experimental import pallas as pl
from jax.experimental.pallas import tpu as pltpu
from jax.experimental.pallas import tpu_sc as plsc

N_EDGES = 6400000
D_EDGE = 4
LANES = 16
ROW = 128
BLK = 2048
NB = N_EDGES // BLK
NC = 2
NS = 16
NW = NC * NS
NB_LO = NB // NW
N_HI = NB - NB_LO * NW

_MESH = plsc.VectorSubcoreMesh(core_axis_name="c", subcore_axis_name="s")
_PARAMS = pltpu.CompilerParams(needs_layout_passes=False,
                               use_tc_tiling_on_sc=False)


def _wid():
    return lax.axis_index("s") * NC + lax.axis_index("c")


def _chunk(w):
    nb = jnp.where(w < N_HI, NB_LO + 1, NB_LO)
    sb = w * NB_LO + jnp.minimum(w, N_HI)
    return sb, nb


def _count_body(mask_hbm, counts_hbm, mask_v, out_v):
    w = _wid()
    sb, nb = _chunk(w)

    def block(k, acc):
        pltpu.sync_copy(mask_hbm.at[pl.ds((sb + k) * (BLK // ROW), BLK // ROW)],
                        mask_v)

        def row(r, acc):
            for q in range(ROW // LANES):
                acc = acc + mask_v[r, pl.ds(q * LANES, LANES)]
            return acc

        return lax.fori_loop(0, BLK // ROW, row, acc)

    acc = lax.fori_loop(0, nb, block, jnp.zeros((LANES,), jnp.int32))
    out_v[...] = jnp.full((LANES,), jnp.sum(acc), jnp.int32)
    pltpu.sync_copy(out_v.at[pl.ds(0, 8)], counts_hbm.at[pl.ds(w * 8, 8)])


def _part_body(mask_hbm, ei0_hbm, ei1_hbm, attr_hbm, counts_hbm,
               oei0_hbm, oei1_hbm, oattr_hbm, nm_hbm,
               mask_v, ei0_v, ei1_v, attr_v, dest_v, destf_v, dest4_v,
               cnt_v, nm_v, sem_in, sem_out):
    w = _wid()
    sb, nb = _chunk(w)

    pltpu.sync_copy(counts_hbm, cnt_v)
    iota = lax.iota(jnp.int32, LANES)
    m_off = jnp.int32(0)
    total = jnp.int32(0)
    for h in range(NW // LANES):
        ids = iota + h * LANES
        vec = plsc.load_gather(cnt_v, [ids * 8])
        m_off = m_off + jnp.sum(jnp.where(ids < w, vec, 0))
        total = total + jnp.sum(vec)
    n_kept = N_EDGES - total

    @pl.when(w == 0)
    def _():
        nm_v[...] = jnp.full((LANES,), total, jnp.int32)
        pltpu.sync_copy(nm_v.at[pl.ds(0, 8)], nm_hbm)

    rows_per_blk = BLK // ROW
    arows_per_blk = BLK * D_EDGE // ROW
    c4 = iota // 4
    ccol = iota % 4

    def block(k, m_run):
        base_row = (sb + k) * rows_per_blk
        base_edge = base_row * ROW
        pltpu.sync_copy(mask_hbm.at[pl.ds(base_row, rows_per_blk)], mask_v)
        din0 = pltpu.async_copy(ei0_hbm.at[pl.ds(base_row, rows_per_blk)],
                                ei0_v, sem_in)
        din1 = pltpu.async_copy(ei1_hbm.at[pl.ds(base_row, rows_per_blk)],
                                ei1_v, sem_in)
        din2 = pltpu.async_copy(
            attr_hbm.at[pl.ds(base_row * D_EDGE, arows_per_blk)],
            attr_v, sem_in)

        for r in range(rows_per_blk):
            def qbody(q, mr, r=r):
                mvec = mask_v[r, pl.ds(q * LANES, LANES)]
                excl = plsc.cumsum(mvec) - mvec
                before = mr + excl
                gi = base_edge + r * ROW + q * LANES + iota
                dest = jnp.where(mvec == 1, n_kept + before, gi - before)
                dest_v[r, pl.ds(q * LANES, LANES)] = dest
                destf_v[pl.ds(r * ROW + q * LANES, LANES)] = dest
                return mr + jnp.sum(mvec)

            m_run = lax.fori_loop(0, ROW // LANES, qbody, m_run)

        def wbody(wr, _):
            for u in range(8):
                p = plsc.load_gather(destf_v, [32 * wr + 4 * u + c4])
                dest4_v[wr, pl.ds(u * LANES, LANES)] = p * 4 + ccol
            return 0

        lax.fori_loop(0, arows_per_blk, wbody, 0)

        din0.wait()
        din1.wait()
        din2.wait()

        outs = []
        for r in range(rows_per_blk):
            idx = dest_v.at[r]
            outs.append(pltpu.async_copy(ei0_v.at[r], oei0_hbm.at[idx],
                                         sem_out))
            outs.append(pltpu.async_copy(ei1_v.at[r], oei1_hbm.at[idx],
                                         sem_out))
        for r in range(arows_per_blk):
            outs.append(pltpu.async_copy(attr_v.at[r],
                                         oattr_hbm.at[dest4_v.at[r]],
                                         sem_out))
        for d in outs:
            d.wait()
        return m_run

    lax.fori_loop(0, nb, block, m_off)


@jax.jit
def kernel(edge_index, edge_attr, mask):
    maski = mask.astype(jnp.int32).reshape(N_EDGES // ROW, ROW)
    ei0 = edge_index[0].reshape(N_EDGES // ROW, ROW)
    ei1 = edge_index[1].reshape(N_EDGES // ROW, ROW)
    attr = edge_attr.reshape(N_EDGES * D_EDGE // ROW, ROW)

    counts = pl.kernel(
        _count_body,
        out_type=jax.ShapeDtypeStruct((NW * 8,), jnp.int32),
        mesh=_MESH,
        compiler_params=_PARAMS,
        scratch_types=[
            pltpu.VMEM((BLK // ROW, ROW), jnp.int32),
            pltpu.VMEM((LANES,), jnp.int32),
        ],
    )(maski)

    oei0, oei1, oattr, nm = pl.kernel(
        _part_body,
        out_type=(
            jax.ShapeDtypeStruct((N_EDGES,), jnp.int32),
            jax.ShapeDtypeStruct((N_EDGES,), jnp.int32),
            jax.ShapeDtypeStruct((N_EDGES * D_EDGE,), jnp.float32),
            jax.ShapeDtypeStruct((8,), jnp.int32),
        ),
        mesh=_MESH,
        compiler_params=_PARAMS,
        scratch_types=[
            pltpu.VMEM((BLK // ROW, ROW), jnp.int32),
            pltpu.VMEM((BLK // ROW, ROW), jnp.int32),
            pltpu.VMEM((BLK // ROW, ROW), jnp.int32),
            pltpu.VMEM((BLK * D_EDGE // ROW, ROW), jnp.float32),
            pltpu.VMEM((BLK // ROW, ROW), jnp.int32),
            pltpu.VMEM((BLK,), jnp.int32),
            pltpu.VMEM((BLK * D_EDGE // ROW, ROW), jnp.int32),
            pltpu.VMEM((NW * 8,), jnp.int32),
            pltpu.VMEM((LANES,), jnp.int32),
            pltpu.SemaphoreType.DMA,
            pltpu.SemaphoreType.DMA,
        ],
    )(maski, ei0, ei1, attr, counts)

    part_edge_index = jnp.stack([oei0, oei1])
    return part_edge_index, oattr.reshape(N_EDGES, D_EDGE), nm[0]
